# baseline (speedup 1.0000x reference)
import functools

import jax
import jax.numpy as jnp
from jax import lax
from jax.experimental import pallas as pl
from jax.experimental.pallas import tpu as pltpu
from jax.experimental.pallas import tpu_sc as plsc

P = 8
PP = P * P
H = 16
B = 1024
C = 96
CC = 16
NSLAB = 18 * 18
NW = 32
MAXS = (NSLAB + NW - 1) // NW


def _tables(topW, botW, leftW, rightW, tlW, trW, blW, brW):
    wv = [2.0 * jnp.tanh(w / 2.0) for w in
          (topW, botW, leftW, rightW, tlW, trW, blW, brW)]
    one = jnp.ones((C,), jnp.float32)
    zeroc = jnp.zeros((C,), jnp.float32)
    del zeroc
    w1 = jnp.stack(wv + [one])
    w1 = jnp.broadcast_to(w1.reshape(9, 12, 8, 1), (9, 12, 8, 16))
    w1 = w1.reshape(9, 12, 128)
    b = jnp.arange(64)
    r = b // P
    c = b % P
    mT = (r > 0).astype(jnp.float32)
    mB = (r < P - 1).astype(jnp.float32)
    mL = (c > 0).astype(jnp.float32)
    mR = (c < P - 1).astype(jnp.float32)
    m = jnp.stack([mT, mB, mL, mR, mT * mL, mT * mR, mB * mL, mB * mR,
                   jnp.ones((64,), jnp.float32)])
    m = m.reshape(9, 4, 16)
    return w1, m


def _sc_kernel(x_hbm, w1_hbm, m_hbm, out_hbm,
               bufa, bufb, w1_v, m_v):
    wid = lax.axis_index("s") * 2 + lax.axis_index("c")
    pltpu.sync_copy(w1_hbm, w1_v)
    pltpu.sync_copy(m_hbm, m_v)

    def slab_loop(t, carry0):
        sid = wid + NW * t

        @pl.when(sid < NSLAB)
        def _():
            u = sid // 18
            v = sid % 18
            interior = jnp.logical_and(
                jnp.logical_and(u > 0, u < 17),
                jnp.logical_and(v > 0, v < 17))
            ty = jnp.where(
                u == 0, jnp.where(v == 0, 4, jnp.where(v == 17, 5, 0)),
                jnp.where(
                    u == 17, jnp.where(v == 0, 6, jnp.where(v == 17, 7, 1)),
                    jnp.where(v == 0, 2, jnp.where(v == 17, 3, 8))))
            hwA = jnp.where(
                ty == 0, v - 1,
                jnp.where(ty == 1, 240 + v - 1,
                jnp.where(ty == 2, 16 * (u - 1),
                jnp.where(ty == 3, 16 * (u - 1) + 15,
                jnp.where(ty == 4, 0,
                jnp.where(ty == 5, 15,
                jnp.where(ty == 6, 240,
                jnp.where(ty == 7, 255, 16 * (u - 1) + (v - 1)))))))))
            hwB = jnp.where(
                ty == 0, 16 + v - 1,
                jnp.where(ty == 1, 224 + v - 1,
                jnp.where(ty == 2, 16 * (u - 1) + 1,
                jnp.where(ty == 3, 16 * (u - 1) + 14, hwA))))

            def chunk_loop(cc, carry1):
                c0 = cc * CC

                @pl.when(interior)
                def _():
                    pltpu.sync_copy(x_hbm.at[pl.ds(c0, CC), hwA, :], bufa)
                    pltpu.sync_copy(bufa, out_hbm.at[sid, pl.ds(c0, CC), :])

                @pl.when(jnp.logical_not(interior))
                def _():
                    pltpu.sync_copy(x_hbm.at[pl.ds(c0, CC), hwA, :], bufa)
                    pltpu.sync_copy(x_hbm.at[pl.ds(c0, CC), hwB, :], bufb)

                    ef = jnp.where(ty < 4, 1.0, 0.0).astype(jnp.float32)

                    def body(i, carry):
                        ci = i // (B // 16)
                        j = i % (B // 16)
                        a = bufa[ci, pl.ds(16 * j, 16)]
                        bb = bufb[ci, pl.ds(16 * j, 16)]
                        cg = c0 + ci
                        wa = w1_v[ty, cg // 8, pl.ds((cg % 8) * 16, 16)]
                        wb = (1.0 - wa) * ef
                        mm = m_v[ty, j % 4, :]
                        bufa[ci, pl.ds(16 * j, 16)] = mm * (wa * a + wb * bb)
                        return carry

                    lax.fori_loop(0, CC * (B // 16), body, 0)
                    pltpu.sync_copy(bufa, out_hbm.at[sid, pl.ds(c0, CC), :])

                return carry1

            lax.fori_loop(0, C // CC, chunk_loop, 0)

        return carry0

    lax.fori_loop(0, MAXS, slab_loop, 0)


def kernel(x, topW, botW, leftW, rightW, topleftW, toprightW, botleftW,
           botrightW, padding, num_patches, scaling_factor):
    b, Cx, ph, pw = x.shape
    x3 = jnp.transpose(x, (1, 2, 3, 0)).reshape(Cx, ph * pw, b)
    w1, m = _tables(topW, botW, leftW, rightW,
                    topleftW, toprightW, botleftW, botrightW)

    mesh = plsc.VectorSubcoreMesh(core_axis_name="c", subcore_axis_name="s")
    run = functools.partial(
        pl.kernel,
        out_type=jax.ShapeDtypeStruct((NSLAB, Cx, b), x.dtype),
        mesh=mesh,
        scratch_types=[
            pltpu.VMEM((CC, b), jnp.float32),
            pltpu.VMEM((CC, b), jnp.float32),
            pltpu.VMEM((9, 12, 128), jnp.float32),
            pltpu.VMEM((9, 4, 16), jnp.float32),
        ],
    )(_sc_kernel)
    out3 = run(x3, w1, m)
    out4 = out3.reshape(ph + 2, pw + 2, Cx, b)
    return jnp.transpose(out4, (3, 2, 0, 1))


# speedup vs baseline: 1.3010x; 1.3010x over previous
import functools

import jax
import jax.numpy as jnp
from jax import lax
from jax.experimental import pallas as pl
from jax.experimental.pallas import tpu as pltpu
from jax.experimental.pallas import tpu_sc as plsc

P = 8
PP = P * P
H = 16
B = 1024
C = 96
CC = 48
NSLAB = 18 * 18
NW = 32
MAXS = (NSLAB + NW - 1) // NW


def _tables(topW, botW, leftW, rightW, tlW, trW, blW, brW):
    wv = [2.0 * jnp.tanh(w / 2.0) for w in
          (topW, botW, leftW, rightW, tlW, trW, blW, brW)]
    one = jnp.ones((C,), jnp.float32)
    zeroc = jnp.zeros((C,), jnp.float32)
    del zeroc
    w1 = jnp.stack(wv + [one])
    w1 = jnp.broadcast_to(w1.reshape(9, 12, 8, 1), (9, 12, 8, 16))
    w1 = w1.reshape(9, 12, 128)
    b = jnp.arange(64)
    r = b // P
    c = b % P
    mT = (r > 0).astype(jnp.float32)
    mB = (r < P - 1).astype(jnp.float32)
    mL = (c > 0).astype(jnp.float32)
    mR = (c < P - 1).astype(jnp.float32)
    m = jnp.stack([mT, mB, mL, mR, mT * mL, mT * mR, mB * mL, mB * mR,
                   jnp.ones((64,), jnp.float32)])
    m = m.reshape(9, 4, 16)
    return w1, m


def _sc_kernel(x_hbm, w1_hbm, m_hbm, out_hbm,
               bufa, bufb, w1_v, m_v):
    wid = lax.axis_index("s") * 2 + lax.axis_index("c")
    pltpu.sync_copy(w1_hbm, w1_v)
    pltpu.sync_copy(m_hbm, m_v)

    def slab_loop(t, carry0):
        sid = wid + NW * t

        @pl.when(sid < NSLAB)
        def _():
            u = sid // 18
            v = sid % 18
            interior = jnp.logical_and(
                jnp.logical_and(u > 0, u < 17),
                jnp.logical_and(v > 0, v < 17))
            ty = jnp.where(
                u == 0, jnp.where(v == 0, 4, jnp.where(v == 17, 5, 0)),
                jnp.where(
                    u == 17, jnp.where(v == 0, 6, jnp.where(v == 17, 7, 1)),
                    jnp.where(v == 0, 2, jnp.where(v == 17, 3, 8))))
            hwA = jnp.where(
                ty == 0, v - 1,
                jnp.where(ty == 1, 240 + v - 1,
                jnp.where(ty == 2, 16 * (u - 1),
                jnp.where(ty == 3, 16 * (u - 1) + 15,
                jnp.where(ty == 4, 0,
                jnp.where(ty == 5, 15,
                jnp.where(ty == 6, 240,
                jnp.where(ty == 7, 255, 16 * (u - 1) + (v - 1)))))))))
            hwB = jnp.where(
                ty == 0, 16 + v - 1,
                jnp.where(ty == 1, 224 + v - 1,
                jnp.where(ty == 2, 16 * (u - 1) + 1,
                jnp.where(ty == 3, 16 * (u - 1) + 14, hwA))))

            def chunk_loop(cc, carry1):
                c0 = cc * CC

                @pl.when(interior)
                def _():
                    pltpu.sync_copy(x_hbm.at[pl.ds(c0, CC), hwA, :], bufa)
                    pltpu.sync_copy(bufa, out_hbm.at[sid, pl.ds(c0, CC), :])

                @pl.when(jnp.logical_not(interior))
                def _():
                    pltpu.sync_copy(x_hbm.at[pl.ds(c0, CC), hwA, :], bufa)
                    pltpu.sync_copy(x_hbm.at[pl.ds(c0, CC), hwB, :], bufb)

                    ef = jnp.where(ty < 4, 1.0, 0.0).astype(jnp.float32)

                    def row_body(ci, carry):
                        cg = c0 + ci
                        wa = w1_v[ty, cg // 8, pl.ds((cg % 8) * 16, 16)]
                        wb = (1.0 - wa) * ef

                        def body(j, carry2):
                            a = bufa[ci, pl.ds(16 * j, 16)]
                            bb = bufb[ci, pl.ds(16 * j, 16)]
                            mm = m_v[ty, j % 4, :]
                            bufa[ci, pl.ds(16 * j, 16)] = mm * (wa * a + wb * bb)
                            return carry2

                        lax.fori_loop(0, B // 16, body, 0)
                        return carry

                    lax.fori_loop(0, CC, row_body, 0)
                    pltpu.sync_copy(bufa, out_hbm.at[sid, pl.ds(c0, CC), :])

                return carry1

            lax.fori_loop(0, C // CC, chunk_loop, 0)

        return carry0

    lax.fori_loop(0, MAXS, slab_loop, 0)


def kernel(x, topW, botW, leftW, rightW, topleftW, toprightW, botleftW,
           botrightW, padding, num_patches, scaling_factor):
    b, Cx, ph, pw = x.shape
    x3 = jnp.transpose(x, (1, 2, 3, 0)).reshape(Cx, ph * pw, b)
    w1, m = _tables(topW, botW, leftW, rightW,
                    topleftW, toprightW, botleftW, botrightW)

    mesh = plsc.VectorSubcoreMesh(core_axis_name="c", subcore_axis_name="s")
    run = functools.partial(
        pl.kernel,
        out_type=jax.ShapeDtypeStruct((NSLAB, Cx, b), x.dtype),
        mesh=mesh,
        scratch_types=[
            pltpu.VMEM((CC, b), jnp.float32),
            pltpu.VMEM((CC, b), jnp.float32),
            pltpu.VMEM((9, 12, 128), jnp.float32),
            pltpu.VMEM((9, 4, 16), jnp.float32),
        ],
    )(_sc_kernel)
    out3 = run(x3, w1, m)
    out4 = out3.reshape(ph + 2, pw + 2, Cx, b)
    return jnp.transpose(out4, (3, 2, 0, 1))


# speedup vs baseline: 1.3312x; 1.0232x over previous
import functools

import jax
import jax.numpy as jnp
from jax import lax
from jax.experimental import pallas as pl
from jax.experimental.pallas import tpu as pltpu
from jax.experimental.pallas import tpu_sc as plsc

P = 8
PP = P * P
H = 16
B = 1024
C = 96
CC = 48
NSLAB = 18 * 18
NW = 32
MAXS = (NSLAB + NW - 1) // NW


def _tables(topW, botW, leftW, rightW, tlW, trW, blW, brW):
    wv = [2.0 * jnp.tanh(w / 2.0) for w in
          (topW, botW, leftW, rightW, tlW, trW, blW, brW)]
    one = jnp.ones((C,), jnp.float32)
    zeroc = jnp.zeros((C,), jnp.float32)
    del zeroc
    w1 = jnp.stack(wv + [one])
    w1 = jnp.broadcast_to(w1.reshape(9, 12, 8, 1), (9, 12, 8, 16))
    w1 = w1.reshape(9, 12, 128)
    b = jnp.arange(64)
    r = b // P
    c = b % P
    mT = (r > 0).astype(jnp.float32)
    mB = (r < P - 1).astype(jnp.float32)
    mL = (c > 0).astype(jnp.float32)
    mR = (c < P - 1).astype(jnp.float32)
    m = jnp.stack([mT, mB, mL, mR, mT * mL, mT * mR, mB * mL, mB * mR,
                   jnp.ones((64,), jnp.float32)])
    m = m.reshape(9, 4, 16)
    return w1, m


def _sc_kernel(x_hbm, w1_hbm, m_hbm, out_hbm,
               bufa, bufb, w1_v, m_v, sem_a, sem_b):
    wid = lax.axis_index("s") * 2 + lax.axis_index("c")
    pltpu.sync_copy(w1_hbm, w1_v)
    pltpu.sync_copy(m_hbm, m_v)

    def slab_loop(t, carry0):
        sid = wid + NW * t

        @pl.when(sid < NSLAB)
        def _():
            u = sid // 18
            v = sid % 18
            interior = jnp.logical_and(
                jnp.logical_and(u > 0, u < 17),
                jnp.logical_and(v > 0, v < 17))
            ty = jnp.where(
                u == 0, jnp.where(v == 0, 4, jnp.where(v == 17, 5, 0)),
                jnp.where(
                    u == 17, jnp.where(v == 0, 6, jnp.where(v == 17, 7, 1)),
                    jnp.where(v == 0, 2, jnp.where(v == 17, 3, 8))))
            hwA = jnp.where(
                ty == 0, v - 1,
                jnp.where(ty == 1, 240 + v - 1,
                jnp.where(ty == 2, 16 * (u - 1),
                jnp.where(ty == 3, 16 * (u - 1) + 15,
                jnp.where(ty == 4, 0,
                jnp.where(ty == 5, 15,
                jnp.where(ty == 6, 240,
                jnp.where(ty == 7, 255, 16 * (u - 1) + (v - 1)))))))))
            hwB = jnp.where(
                ty == 0, 16 + v - 1,
                jnp.where(ty == 1, 224 + v - 1,
                jnp.where(ty == 2, 16 * (u - 1) + 1,
                jnp.where(ty == 3, 16 * (u - 1) + 14, hwA))))

            @pl.when(interior)
            def _():
                ha = pltpu.async_copy(x_hbm.at[pl.ds(0, CC), hwA, :],
                                      bufa, sem_a)
                hb = pltpu.async_copy(x_hbm.at[pl.ds(CC, CC), hwA, :],
                                      bufb, sem_b)
                ha.wait()
                hb.wait()
                oa = pltpu.async_copy(bufa, out_hbm.at[sid, pl.ds(0, CC), :],
                                      sem_a)
                ob = pltpu.async_copy(bufb, out_hbm.at[sid, pl.ds(CC, CC), :],
                                      sem_b)
                oa.wait()
                ob.wait()

            @pl.when(jnp.logical_not(interior))
            def _():
                def chunk_loop(cc, carry1):
                    c0 = cc * CC
                    ha = pltpu.async_copy(x_hbm.at[pl.ds(c0, CC), hwA, :],
                                          bufa, sem_a)
                    hb = pltpu.async_copy(x_hbm.at[pl.ds(c0, CC), hwB, :],
                                          bufb, sem_b)
                    ha.wait()
                    hb.wait()

                    ef = jnp.where(ty < 4, 1.0, 0.0).astype(jnp.float32)

                    def row_body(ci, carry):
                        cg = c0 + ci
                        wa = w1_v[ty, cg // 8, pl.ds((cg % 8) * 16, 16)]
                        wb = (1.0 - wa) * ef

                        def body(j, carry2):
                            a = bufa[ci, pl.ds(16 * j, 16)]
                            bb = bufb[ci, pl.ds(16 * j, 16)]
                            mm = m_v[ty, j % 4, :]
                            bufa[ci, pl.ds(16 * j, 16)] = mm * (wa * a + wb * bb)
                            return carry2

                        lax.fori_loop(0, B // 16, body, 0)
                        return carry

                    lax.fori_loop(0, CC, row_body, 0)
                    pltpu.sync_copy(bufa, out_hbm.at[sid, pl.ds(c0, CC), :])
                    return carry1

                lax.fori_loop(0, C // CC, chunk_loop, 0)

        return carry0

    lax.fori_loop(0, MAXS, slab_loop, 0)


def kernel(x, topW, botW, leftW, rightW, topleftW, toprightW, botleftW,
           botrightW, padding, num_patches, scaling_factor):
    b, Cx, ph, pw = x.shape
    x3 = jnp.transpose(x, (1, 2, 3, 0)).reshape(Cx, ph * pw, b)
    w1, m = _tables(topW, botW, leftW, rightW,
                    topleftW, toprightW, botleftW, botrightW)

    mesh = plsc.VectorSubcoreMesh(core_axis_name="c", subcore_axis_name="s")
    run = functools.partial(
        pl.kernel,
        out_type=jax.ShapeDtypeStruct((NSLAB, Cx, b), x.dtype),
        mesh=mesh,
        scratch_types=[
            pltpu.VMEM((CC, b), jnp.float32),
            pltpu.VMEM((CC, b), jnp.float32),
            pltpu.VMEM((9, 12, 128), jnp.float32),
            pltpu.VMEM((9, 4, 16), jnp.float32),
            pltpu.SemaphoreType.DMA,
            pltpu.SemaphoreType.DMA,
        ],
    )(_sc_kernel)
    out3 = run(x3, w1, m)
    out4 = out3.reshape(ph + 2, pw + 2, Cx, b)
    return jnp.transpose(out4, (3, 2, 0, 1))


# speedup vs baseline: 4.0164x; 3.0170x over previous
import jax
import jax.numpy as jnp
from jax import lax
from jax.experimental import pallas as pl

P = 8
PP = P * P
H = 16
C_BLK = 8
R_OUT = 9
R_IN = 8


def _halo_kernel(tw_ref, bw_ref, lw_ref, rw_ref, tlw_ref, trw_ref,
                 blw_ref, brw_ref, x_ref, out_ref):
    k = pl.program_id(0)

    bm = lax.broadcasted_iota(jnp.int32, (1, 1, C_BLK, 1024), 3)
    im = bm % PP
    r = im // P
    c = im % P
    mT = r > 0
    mB = r < P - 1
    mL = c > 0
    mR = c < P - 1

    tW = tw_ref[...]
    bW = bw_ref[...]
    lW = lw_ref[...]
    rW = rw_ref[...]
    zero = jnp.zeros((), jnp.float32)

    xt = jnp.transpose(x_ref[...], (1, 2, 0, 3))

    def interior(j0, rr):
        xr = xt[rr:rr+1]
        c0 = jnp.where(mL, lW * xr[:, 0:1] + (1.0 - lW) * xr[:, 1:2], zero)
        c17 = jnp.where(mR, rW * xr[:, H-1:H] + (1.0 - rW) * xr[:, H-2:H-1],
                        zero)
        out_ref[j0:j0+1] = jnp.concatenate([c0, xr, c17], axis=1)

    @pl.when(k == 0)
    def _():
        x0 = xt[0:1]
        x1 = xt[1:2]
        mid = jnp.where(mT, tW * x0 + (1.0 - tW) * x1, zero)
        c0 = jnp.where(mT & mL, tlw_ref[...] * x0[:, 0:1], zero)
        c17 = jnp.where(mT & mR, trw_ref[...] * x0[:, H-1:H], zero)
        out_ref[0:1] = jnp.concatenate([c0, mid, c17], axis=1)
        for j0 in range(1, R_OUT):
            interior(j0, j0 - 1)

    @pl.when(k == 1)
    def _():
        for j0 in range(0, R_OUT - 1):
            interior(j0, j0)
        x15 = xt[R_IN-1:R_IN]
        x14 = xt[R_IN-2:R_IN-1]
        mid = jnp.where(mB, bW * x15 + (1.0 - bW) * x14, zero)
        c0 = jnp.where(mB & mL, blw_ref[...] * x15[:, 0:1], zero)
        c17 = jnp.where(mB & mR, brw_ref[...] * x15[:, H-1:H], zero)
        out_ref[R_OUT-1:R_OUT] = jnp.concatenate([c0, mid, c17], axis=1)


def kernel(x, topW, botW, leftW, rightW, topleftW, toprightW, botleftW,
           botrightW, padding, num_patches, scaling_factor):
    b, C, ph, pw = x.shape
    xv = jnp.transpose(x, (1, 2, 3, 0))
    ws = [jnp.broadcast_to((2.0 * jnp.tanh(w / 2.0)).reshape(1, 1, C, 1),
                           (1, 1, C, b))
          for w in (topW, botW, leftW, rightW,
                    topleftW, toprightW, botleftW, botrightW)]

    w_spec = pl.BlockSpec((1, 1, C_BLK, b), lambda k, j: (0, 0, j, 0))
    out_t = pl.pallas_call(
        _halo_kernel,
        grid=(2, C // C_BLK),
        in_specs=[w_spec] * 8 + [
            pl.BlockSpec((C_BLK, R_IN, pw, b), lambda k, j: (j, k, 0, 0)),
        ],
        out_specs=pl.BlockSpec((R_OUT, pw + 2, C_BLK, b),
                               lambda k, j: (k, 0, j, 0)),
        out_shape=jax.ShapeDtypeStruct((ph + 2, pw + 2, C, b), x.dtype),
    )(*ws, xv)
    return jnp.transpose(out_t, (3, 2, 0, 1))


# speedup vs baseline: 4.0871x; 1.0176x over previous
import jax
import jax.numpy as jnp
from jax import lax
from jax.experimental import pallas as pl

P = 8
PP = P * P
H = 16
C_BLK = 16
R_OUT = 9
R_IN = 8


def _halo_kernel(tw_ref, bw_ref, lw_ref, rw_ref, tlw_ref, trw_ref,
                 blw_ref, brw_ref, x_ref, out_ref):
    k = pl.program_id(0)

    bm = lax.broadcasted_iota(jnp.int32, (1, 1, C_BLK, 1024), 3)
    im = bm % PP
    r = im // P
    c = im % P
    mT = r > 0
    mB = r < P - 1
    mL = c > 0
    mR = c < P - 1

    tW = tw_ref[...]
    bW = bw_ref[...]
    lW = lw_ref[...]
    rW = rw_ref[...]
    zero = jnp.zeros((), jnp.float32)

    xt = jnp.transpose(x_ref[...], (1, 2, 0, 3))

    def interior(j0, rr):
        xr = xt[rr:rr+1]
        c0 = jnp.where(mL, lW * xr[:, 0:1] + (1.0 - lW) * xr[:, 1:2], zero)
        c17 = jnp.where(mR, rW * xr[:, H-1:H] + (1.0 - rW) * xr[:, H-2:H-1],
                        zero)
        out_ref[j0:j0+1] = jnp.concatenate([c0, xr, c17], axis=1)

    @pl.when(k == 0)
    def _():
        x0 = xt[0:1]
        x1 = xt[1:2]
        mid = jnp.where(mT, tW * x0 + (1.0 - tW) * x1, zero)
        c0 = jnp.where(mT & mL, tlw_ref[...] * x0[:, 0:1], zero)
        c17 = jnp.where(mT & mR, trw_ref[...] * x0[:, H-1:H], zero)
        out_ref[0:1] = jnp.concatenate([c0, mid, c17], axis=1)
        for j0 in range(1, R_OUT):
            interior(j0, j0 - 1)

    @pl.when(k == 1)
    def _():
        for j0 in range(0, R_OUT - 1):
            interior(j0, j0)
        x15 = xt[R_IN-1:R_IN]
        x14 = xt[R_IN-2:R_IN-1]
        mid = jnp.where(mB, bW * x15 + (1.0 - bW) * x14, zero)
        c0 = jnp.where(mB & mL, blw_ref[...] * x15[:, 0:1], zero)
        c17 = jnp.where(mB & mR, brw_ref[...] * x15[:, H-1:H], zero)
        out_ref[R_OUT-1:R_OUT] = jnp.concatenate([c0, mid, c17], axis=1)


def kernel(x, topW, botW, leftW, rightW, topleftW, toprightW, botleftW,
           botrightW, padding, num_patches, scaling_factor):
    b, C, ph, pw = x.shape
    xv = jnp.transpose(x, (1, 2, 3, 0))
    ws = [jnp.broadcast_to((2.0 * jnp.tanh(w / 2.0)).reshape(1, 1, C, 1),
                           (1, 1, C, b))
          for w in (topW, botW, leftW, rightW,
                    topleftW, toprightW, botleftW, botrightW)]

    w_spec = pl.BlockSpec((1, 1, C_BLK, b), lambda k, j: (0, 0, j, 0))
    out_t = pl.pallas_call(
        _halo_kernel,
        grid=(2, C // C_BLK),
        in_specs=[w_spec] * 8 + [
            pl.BlockSpec((C_BLK, R_IN, pw, b), lambda k, j: (j, k, 0, 0)),
        ],
        out_specs=pl.BlockSpec((R_OUT, pw + 2, C_BLK, b),
                               lambda k, j: (k, 0, j, 0)),
        out_shape=jax.ShapeDtypeStruct((ph + 2, pw + 2, C, b), x.dtype),
    )(*ws, xv)
    return jnp.transpose(out_t, (3, 2, 0, 1))
